# baseline (device time: 29957 ns/iter reference)
import os

import jax
import jax.numpy as jnp
from jax import lax
from jax.experimental import pallas as pl
from jax.experimental.pallas import tpu as pltpu

N_DEV = 16
N_PLANE = 4
N_Z = 4
B, SQ, SKV, HQ_PER, DH = 2, 128, 128, 4, 64
D_MODEL = 512
ROWS = B * SQ
QR = ROWS // N_PLANE
CH = QR // N_Z


def _compute(x_ref, wq_ref, k_ref, v_ref, wo_ref, acc_ref):
    xb = x_ref[...].reshape(ROWS, D_MODEL).astype(jnp.bfloat16)
    wq = wq_ref[...].astype(jnp.bfloat16)
    q = lax.dot_general(xb, wq, (((1,), (0,)), ((), ())),
                        preferred_element_type=jnp.float32)
    q = (q * 0.125).astype(jnp.bfloat16)

    ctx_rows = []
    for b in range(B):
        heads = []
        for h in range(HQ_PER):
            qb = q[b * SQ:(b + 1) * SQ, h * DH:(h + 1) * DH]
            kb = k_ref[b, h]
            vb = v_ref[b, h]
            s = lax.dot_general(qb, kb, (((1,), (1,)), ((), ())),
                                preferred_element_type=jnp.float32)
            qi = lax.broadcasted_iota(jnp.int32, (SQ, SKV), 0) // 64
            kj = lax.broadcasted_iota(jnp.int32, (SQ, SKV), 1) // 64
            s = jnp.where(kj <= qi, s, -1e9)
            m = jnp.max(s, axis=1, keepdims=True)
            w = jnp.exp(s - m)
            w = w / jnp.sum(w, axis=1, keepdims=True)
            heads.append(lax.dot_general(w.astype(jnp.bfloat16), vb,
                                         (((1,), (0,)), ((), ())),
                                         preferred_element_type=jnp.float32))
        ctx_rows.append(jnp.concatenate(heads, axis=1))
    ctx = jnp.concatenate(ctx_rows, axis=0).astype(jnp.bfloat16)
    acc_ref[...] = lax.dot_general(ctx, wo_ref[...].astype(jnp.bfloat16),
                                   (((1,), (0,)), ((), ())),
                                   preferred_element_type=jnp.float32
                                   ).astype(jnp.bfloat16)


def kernel(x, Wq, K_ext, V_ext, Wo):
    my = lax.axis_index("i")
    h0 = my * HQ_PER
    K_sl = jnp.transpose(
        lax.dynamic_slice_in_dim(K_ext, h0, HQ_PER, 2).astype(jnp.bfloat16),
        (0, 2, 1, 3))
    V_sl = jnp.transpose(
        lax.dynamic_slice_in_dim(V_ext, h0, HQ_PER, 2).astype(jnp.bfloat16),
        (0, 2, 1, 3))

    _kmode = os.environ.get("KMODE", "full")

    def body(x_ref, wq_ref, k_ref, v_ref, wo_ref, out_ref,
             acc_ref, qa_ref, bufA, bufB,
             sA_send, sA_recv, sB_send, sB_recv,
             sC_send, sC_recv, sD_send, sD_recv):
        my_i = lax.axis_index("i")
        p = lax.rem(my_i, N_PLANE)
        z = my_i // N_PLANE

        if _kmode == "comm":
            acc_ref[...] = x_ref[...].reshape(ROWS, D_MODEL).astype(jnp.bfloat16)
        else:
            _compute(x_ref, wq_ref, k_ref, v_ref, wo_ref, acc_ref)
        if _kmode == "compute":
            out_ref[...] = acc_ref[...]
            return

        a_sends = []
        for j in range(1, N_PLANE):
            pp = lax.rem(p + j, N_PLANE)
            rdma = pltpu.make_async_remote_copy(
                src_ref=acc_ref.at[pl.ds(pp * QR, QR), :],
                dst_ref=bufA.at[p],
                send_sem=sA_send.at[pp],
                recv_sem=sA_recv.at[p],
                device_id=(z * N_PLANE + pp,),
                device_id_type=pl.DeviceIdType.MESH,
            )
            rdma.start()
            a_sends.append(rdma)

        qsum = acc_ref[pl.ds(p * QR, QR), :].astype(jnp.float32)
        for j in range(1, N_PLANE):
            pp = lax.rem(p + j, N_PLANE)
            recv = pltpu.make_async_remote_copy(
                src_ref=acc_ref.at[pl.ds(0, QR), :],
                dst_ref=bufA.at[pp],
                send_sem=sA_send.at[pp],
                recv_sem=sA_recv.at[pp],
                device_id=(z * N_PLANE + pp,),
                device_id_type=pl.DeviceIdType.MESH,
            )
            recv.wait_recv()
            qsum = qsum + bufA[pp].astype(jnp.float32)
        qa_ref[...] = qsum.astype(jnp.bfloat16)
        for rdma in a_sends:
            rdma.wait_send()

        b_sends = []
        for j in range(1, N_Z):
            zz = lax.rem(z + j, N_Z)
            rdma = pltpu.make_async_remote_copy(
                src_ref=qa_ref.at[pl.ds(zz * CH, CH), :],
                dst_ref=bufB.at[z],
                send_sem=sB_send.at[zz],
                recv_sem=sB_recv.at[z],
                device_id=(zz * N_PLANE + p,),
                device_id_type=pl.DeviceIdType.MESH,
            )
            rdma.start()
            b_sends.append(rdma)

        red = qa_ref[pl.ds(z * CH, CH), :].astype(jnp.float32)
        for j in range(1, N_Z):
            zz = lax.rem(z + j, N_Z)
            recv = pltpu.make_async_remote_copy(
                src_ref=qa_ref.at[pl.ds(0, CH), :],
                dst_ref=bufB.at[zz],
                send_sem=sB_send.at[zz],
                recv_sem=sB_recv.at[zz],
                device_id=(zz * N_PLANE + p,),
                device_id_type=pl.DeviceIdType.MESH,
            )
            recv.wait_recv()
            red = red + bufB[zz].astype(jnp.float32)

        r0 = p * QR + z * CH
        out_ref[pl.ds(r0, CH), :] = red.astype(jnp.bfloat16)
        for rdma in b_sends:
            rdma.wait_send()

        c_sends = []
        for j in range(1, N_Z):
            zz = lax.rem(z + j, N_Z)
            rdma = pltpu.make_async_remote_copy(
                src_ref=out_ref.at[pl.ds(r0, CH), :],
                dst_ref=out_ref.at[pl.ds(r0, CH), :],
                send_sem=sC_send.at[zz],
                recv_sem=sC_recv.at[z],
                device_id=(zz * N_PLANE + p,),
                device_id_type=pl.DeviceIdType.MESH,
            )
            rdma.start()
            c_sends.append(rdma)

        for j in range(1, N_Z):
            zz = lax.rem(z + j, N_Z)
            rr = p * QR + zz * CH
            recv = pltpu.make_async_remote_copy(
                src_ref=out_ref.at[pl.ds(0, CH), :],
                dst_ref=out_ref.at[pl.ds(rr, CH), :],
                send_sem=sC_send.at[zz],
                recv_sem=sC_recv.at[zz],
                device_id=(zz * N_PLANE + p,),
                device_id_type=pl.DeviceIdType.MESH,
            )
            recv.wait_recv()
        for rdma in c_sends:
            rdma.wait_send()

        d_sends = []
        for j in range(1, N_PLANE):
            pp = lax.rem(p + j, N_PLANE)
            rdma = pltpu.make_async_remote_copy(
                src_ref=out_ref.at[pl.ds(p * QR, QR), :],
                dst_ref=out_ref.at[pl.ds(p * QR, QR), :],
                send_sem=sD_send.at[pp],
                recv_sem=sD_recv.at[p],
                device_id=(z * N_PLANE + pp,),
                device_id_type=pl.DeviceIdType.MESH,
            )
            rdma.start()
            d_sends.append(rdma)

        for j in range(1, N_PLANE):
            pp = lax.rem(p + j, N_PLANE)
            recv = pltpu.make_async_remote_copy(
                src_ref=out_ref.at[pl.ds(0, QR), :],
                dst_ref=out_ref.at[pl.ds(pp * QR, QR), :],
                send_sem=sD_send.at[pp],
                recv_sem=sD_recv.at[pp],
                device_id=(z * N_PLANE + pp,),
                device_id_type=pl.DeviceIdType.MESH,
            )
            recv.wait_recv()
        for rdma in d_sends:
            rdma.wait_send()

    out = pl.pallas_call(
        body,
        out_shape=jax.ShapeDtypeStruct((ROWS, D_MODEL), jnp.bfloat16),
        in_specs=[pl.BlockSpec(memory_space=pltpu.VMEM)] * 5,
        out_specs=pl.BlockSpec(memory_space=pltpu.VMEM),
        scratch_shapes=[
            pltpu.VMEM((ROWS, D_MODEL), jnp.bfloat16),
            pltpu.VMEM((QR, D_MODEL), jnp.bfloat16),
            pltpu.VMEM((N_PLANE, QR, D_MODEL), jnp.bfloat16),
            pltpu.VMEM((N_Z, CH, D_MODEL), jnp.bfloat16),
            pltpu.SemaphoreType.DMA((N_PLANE,)),
            pltpu.SemaphoreType.DMA((N_PLANE,)),
            pltpu.SemaphoreType.DMA((N_Z,)),
            pltpu.SemaphoreType.DMA((N_Z,)),
            pltpu.SemaphoreType.DMA((N_Z,)),
            pltpu.SemaphoreType.DMA((N_Z,)),
            pltpu.SemaphoreType.DMA((N_PLANE,)),
            pltpu.SemaphoreType.DMA((N_PLANE,)),
        ],
    )(x, Wq, K_sl, V_sl, Wo)
    return out.reshape(B, SQ, D_MODEL)


# device time: 19068 ns/iter; 1.5711x vs baseline; 1.5711x over previous
import os

import jax
import jax.numpy as jnp
from jax import lax
from jax.experimental import pallas as pl
from jax.experimental.pallas import tpu as pltpu

N_DEV = 16
B, SQ, SKV, HQ_PER, DH = 2, 128, 128, 4, 64
D_MODEL = 512
ROWS = B * SQ
CH = ROWS // N_DEV


def _compute(x_ref, wq_ref, k_ref, v_ref, wo_ref, acc_ref):
    xb = x_ref[...].reshape(ROWS, D_MODEL).astype(jnp.bfloat16)
    wq = wq_ref[...].astype(jnp.bfloat16)
    q = lax.dot_general(xb, wq, (((1,), (0,)), ((), ())),
                        preferred_element_type=jnp.float32)
    q = (q * 0.125).astype(jnp.bfloat16)

    ctx_rows = []
    for b in range(B):
        heads = []
        for h in range(HQ_PER):
            qb = q[b * SQ:(b + 1) * SQ, h * DH:(h + 1) * DH]
            kb = k_ref[b, h]
            vb = v_ref[b, h]
            s = lax.dot_general(qb, kb, (((1,), (1,)), ((), ())),
                                preferred_element_type=jnp.float32)
            qi = lax.broadcasted_iota(jnp.int32, (SQ, SKV), 0) // 64
            kj = lax.broadcasted_iota(jnp.int32, (SQ, SKV), 1) // 64
            s = jnp.where(kj <= qi, s, -1e9)
            m = jnp.max(s, axis=1, keepdims=True)
            w = jnp.exp(s - m)
            w = w / jnp.sum(w, axis=1, keepdims=True)
            heads.append(lax.dot_general(w.astype(jnp.bfloat16), vb,
                                         (((1,), (0,)), ((), ())),
                                         preferred_element_type=jnp.float32))
        ctx_rows.append(jnp.concatenate(heads, axis=1))
    ctx = jnp.concatenate(ctx_rows, axis=0).astype(jnp.bfloat16)
    acc_ref[...] = lax.dot_general(ctx, wo_ref[...].astype(jnp.bfloat16),
                                   (((1,), (0,)), ((), ())),
                                   preferred_element_type=jnp.float32
                                   ).astype(jnp.bfloat16)


def kernel(x, Wq, K_ext, V_ext, Wo):
    my = lax.axis_index("i")
    h0 = my * HQ_PER
    K_sl = jnp.transpose(
        lax.dynamic_slice_in_dim(K_ext, h0, HQ_PER, 2).astype(jnp.bfloat16),
        (0, 2, 1, 3))
    V_sl = jnp.transpose(
        lax.dynamic_slice_in_dim(V_ext, h0, HQ_PER, 2).astype(jnp.bfloat16),
        (0, 2, 1, 3))

    _kmode = os.environ.get("KMODE", "full")

    def body(x_ref, wq_ref, k_ref, v_ref, wo_ref, out_ref,
             acc_ref, rs_buf, rs_send_sems, rs_recv_sems,
             ag_send_sems, ag_recv_sems):
        my_i = lax.axis_index("i")

        if _kmode in ("comm", "comm1", "barrier"):
            acc_ref[...] = x_ref[...].reshape(ROWS, D_MODEL).astype(jnp.bfloat16)
        else:
            _compute(x_ref, wq_ref, k_ref, v_ref, wo_ref, acc_ref)
        if _kmode == "compute":
            out_ref[...] = acc_ref[...]
            return
        if _kmode == "barrier":
            tgt = lax.rem(my_i + 1, N_DEV)
            rdma = pltpu.make_async_remote_copy(
                src_ref=acc_ref.at[pl.ds(0, CH), :],
                dst_ref=rs_buf.at[0],
                send_sem=rs_send_sems.at[0],
                recv_sem=rs_recv_sems.at[0],
                device_id=(tgt,),
                device_id_type=pl.DeviceIdType.MESH,
            )
            rdma.start()
            rdma.wait_send()
            src_id = lax.rem(my_i - 1 + N_DEV, N_DEV)
            recv = pltpu.make_async_remote_copy(
                src_ref=acc_ref.at[pl.ds(0, CH), :],
                dst_ref=rs_buf.at[0],
                send_sem=rs_send_sems.at[0],
                recv_sem=rs_recv_sems.at[0],
                device_id=(src_id,),
                device_id_type=pl.DeviceIdType.MESH,
            )
            recv.wait_recv()
            out_ref[...] = acc_ref[...]
            return

        rs_sends = []
        for j in range(N_DEV - 1):
            tgt = lax.rem(my_i + j + 1, N_DEV)
            rdma = pltpu.make_async_remote_copy(
                src_ref=acc_ref.at[pl.ds(tgt * CH, CH), :],
                dst_ref=rs_buf.at[my_i],
                send_sem=rs_send_sems.at[tgt],
                recv_sem=rs_recv_sems.at[my_i],
                device_id=(tgt,),
                device_id_type=pl.DeviceIdType.MESH,
            )
            rdma.start()
            rs_sends.append(rdma)

        red = acc_ref[pl.ds(my_i * CH, CH), :].astype(jnp.float32)
        for k in range(N_DEV - 1):
            s_id = lax.rem(my_i + k + 1, N_DEV)
            recv = pltpu.make_async_remote_copy(
                src_ref=acc_ref.at[pl.ds(0, CH), :],
                dst_ref=rs_buf.at[s_id],
                send_sem=rs_send_sems.at[s_id],
                recv_sem=rs_recv_sems.at[s_id],
                device_id=(s_id,),
                device_id_type=pl.DeviceIdType.MESH,
            )
            recv.wait_recv()
            red = red + rs_buf[s_id].astype(jnp.float32)

        out_ref[pl.ds(my_i * CH, CH), :] = red.astype(jnp.bfloat16)

        for rdma in rs_sends:
            rdma.wait_send()

        if _kmode == "comm1":
            return

        ag_sends = []
        for j in range(N_DEV - 1):
            tgt = lax.rem(my_i + j + 1, N_DEV)
            rdma = pltpu.make_async_remote_copy(
                src_ref=out_ref.at[pl.ds(my_i * CH, CH), :],
                dst_ref=out_ref.at[pl.ds(my_i * CH, CH), :],
                send_sem=ag_send_sems.at[tgt],
                recv_sem=ag_recv_sems.at[my_i],
                device_id=(tgt,),
                device_id_type=pl.DeviceIdType.MESH,
            )
            rdma.start()
            ag_sends.append(rdma)

        for k in range(N_DEV - 1):
            s_id = lax.rem(my_i + k + 1, N_DEV)
            recv = pltpu.make_async_remote_copy(
                src_ref=out_ref.at[pl.ds(0, CH), :],
                dst_ref=out_ref.at[pl.ds(s_id * CH, CH), :],
                send_sem=ag_send_sems.at[s_id],
                recv_sem=ag_recv_sems.at[s_id],
                device_id=(s_id,),
                device_id_type=pl.DeviceIdType.MESH,
            )
            recv.wait_recv()

        for rdma in ag_sends:
            rdma.wait_send()

    out = pl.pallas_call(
        body,
        out_shape=jax.ShapeDtypeStruct((ROWS, D_MODEL), jnp.bfloat16),
        in_specs=[pl.BlockSpec(memory_space=pltpu.VMEM)] * 5,
        out_specs=pl.BlockSpec(memory_space=pltpu.VMEM),
        scratch_shapes=[
            pltpu.VMEM((ROWS, D_MODEL), jnp.bfloat16),
            pltpu.VMEM((N_DEV, CH, D_MODEL), jnp.bfloat16),
            pltpu.SemaphoreType.DMA((N_DEV,)),
            pltpu.SemaphoreType.DMA((N_DEV,)),
            pltpu.SemaphoreType.DMA((N_DEV,)),
            pltpu.SemaphoreType.DMA((N_DEV,)),
        ],
    )(x, Wq, K_sl, V_sl, Wo)
    return out.reshape(B, SQ, D_MODEL)
